# baseline (speedup 1.0000x reference)
import functools

import jax
import jax.numpy as jnp
from jax import lax
from jax.experimental import pallas as pl
from jax.experimental.pallas import tpu as pltpu
from jax.experimental.pallas import tpu_sc as plsc

B = 16384
D = 32
NC, NS, L = 2, 16, 16
NW = NC * NS
BPW = B // NW
CH = 128
NCH = BPW // CH
NG = BPW // L


def _mf_body(u_hbm, i_hbm, ub_hbm, uv_hbm, ib_hbm, iv_hbm, g_hbm, out_hbm,
             u_idx, i_idx, vu, vi, bu, bi, outv, gv, sem):
    wid = lax.axis_index("s") * NC + lax.axis_index("c")

    pltpu.sync_copy(u_hbm.at[wid], u_idx)
    pltpu.sync_copy(i_hbm.at[wid], i_idx)
    pltpu.sync_copy(g_hbm, gv)

    copies = []
    for c in range(NCH):
        r = pl.ds(c * CH, CH)
        copies.append(pltpu.async_copy(uv_hbm.at[u_idx.at[c]], vu.at[r], sem))
        copies.append(pltpu.async_copy(iv_hbm.at[i_idx.at[c]], vi.at[r], sem))
        copies.append(pltpu.async_copy(ub_hbm.at[u_idx.at[c]], bu.at[r], sem))
        copies.append(pltpu.async_copy(ib_hbm.at[i_idx.at[c]], bi.at[r], sem))
    for cp in copies:
        cp.wait()

    glob = gv[...]
    lane = lax.iota(jnp.int32, L)

    def group(g, _):
        base = pl.multiple_of(g * L, L)
        row = lane + g * L
        acc = bu[pl.ds(base, L)] + bi[pl.ds(base, L)] + glob
        for d in range(D):
            col = jnp.full((L,), d, jnp.int32)
            acc = acc + plsc.load_gather(vu, [row, col]) * plsc.load_gather(vi, [row, col])
        outv[pl.ds(base, L)] = acc
        return _

    lax.fori_loop(0, NG, group, 0)

    pltpu.sync_copy(outv, out_hbm.at[pl.ds(wid * BPW, BPW)])


@jax.jit
def _mf(u, i, user_bias, user_vec, item_bias, item_vec, glob_bias):
    mesh = plsc.VectorSubcoreMesh(core_axis_name="c", subcore_axis_name="s",
                                  num_cores=NC, num_subcores=NS)
    return pl.kernel(
        _mf_body,
        out_type=jax.ShapeDtypeStruct((B,), jnp.float32),
        mesh=mesh,
        compiler_params=pltpu.CompilerParams(
            needs_layout_passes=False, use_tc_tiling_on_sc=False),
        scratch_types=[
            pltpu.VMEM((NCH, CH), jnp.int32),
            pltpu.VMEM((NCH, CH), jnp.int32),
            pltpu.VMEM((BPW, D), jnp.float32),
            pltpu.VMEM((BPW, D), jnp.float32),
            pltpu.VMEM((BPW,), jnp.float32),
            pltpu.VMEM((BPW,), jnp.float32),
            pltpu.VMEM((BPW,), jnp.float32),
            pltpu.VMEM((L,), jnp.float32),
            pltpu.SemaphoreType.DMA,
        ],
    )(u, i, user_bias, user_vec, item_bias, item_vec, glob_bias)


def kernel(u, i, user_bias, user_vec, item_bias, item_vec, glob_bias):
    u = u.astype(jnp.int32).reshape(NW, NCH, CH)
    i = i.astype(jnp.int32).reshape(NW, NCH, CH)
    glob = jnp.broadcast_to(glob_bias.reshape(1), (L,))
    return _mf(u, i, user_bias, user_vec, item_bias, item_vec, glob)
